# baseline (speedup 1.0000x reference)
import functools

import jax
import jax.numpy as jnp
import numpy as np
from jax import lax
from jax.experimental import pallas as pl
from jax.experimental.pallas import tpu as pltpu
from jax.experimental.pallas import tpu_sc as plsc

_BATCH = 2048
_S = 14
_D = 30
_P = _S * _S
_NW = 32
_BT = 128
_P_SC = 84
_P_TC = _P - _P_SC
_POS_PER_TILE = _P_SC // 2
_CHUNK_POS = 7
_NCHUNK = _POS_PER_TILE // _CHUNK_POS
_LGROUPS = _BT // 16
_PB = 7
_INV_S = 1.0 / _S
_L_COORD = 5.0
_L_NOOBJ = 0.5


def _sqrt_newton(x):
    i = lax.bitcast_convert_type(x, jnp.int32)
    y = lax.bitcast_convert_type(np.int32(0x5F3759DF) - (i >> 1), jnp.float32)
    xh = jnp.float32(0.5) * x
    three_half = jnp.float32(1.5)
    y = y * (three_half - xh * y * y)
    y = y * (three_half - xh * y * y)
    y = y * (three_half - xh * y * y)
    return x * y


def _cell_loss(P, T, sqrt_fn):
    inv_s = jnp.float32(_INV_S)
    half = jnp.float32(0.5)

    def corners(cx, cy, w, h):
        x = cx * inv_s
        y = cy * inv_s
        hw = half * w
        hh = half * h
        return x - hw, y - hh, x + hw, y + hh

    ax1, ay1, ax2, ay2 = corners(P[2], P[3], P[4], P[5])
    bx1, by1, bx2, by2 = corners(P[6], P[7], P[8], P[9])
    tx1, ty1, tx2, ty2 = corners(T[2], T[3], T[4], T[5])
    area_t = (tx2 - tx1) * (ty2 - ty1)

    def iou(x1, y1, x2, y2):
        lx = jnp.maximum(x1, tx1)
        ly = jnp.maximum(y1, ty1)
        rx = jnp.minimum(x2, tx2)
        ry = jnp.minimum(y2, ty2)
        iw = jnp.maximum(rx - lx, jnp.float32(0.0))
        ih = jnp.maximum(ry - ly, jnp.float32(0.0))
        inter = iw * ih
        area = (x2 - x1) * (y2 - y1)
        return inter / (area + area_t - inter + jnp.float32(1e-10))

    iou0 = iou(ax1, ay1, ax2, ay2)
    iou1 = iou(bx1, by1, bx2, by2)
    sel1 = iou1 > iou0
    max_iou = jnp.where(sel1, iou1, iou0)
    conf_a = jnp.where(sel1, P[1], P[0])
    conf_i = jnp.where(sel1, P[0], P[1])
    d_ca = conf_a - max_iou
    contain = d_ca * d_ca
    notcontain = conf_i * conf_i

    spx = jnp.where(sel1, P[6], P[2])
    spy = jnp.where(sel1, P[7], P[3])
    spw = jnp.where(sel1, P[8], P[4])
    sph = jnp.where(sel1, P[9], P[5])
    stx = jnp.where(sel1, T[6], T[2])
    sty = jnp.where(sel1, T[7], T[3])
    stw = jnp.where(sel1, T[8], T[4])
    sth = jnp.where(sel1, T[9], T[5])
    dx = spx - stx
    dy = spy - sty
    two = jnp.float32(2.0)
    loc = (dx * dx + dy * dy
           + (spw + stw - two * sqrt_fn(spw * stw))
           + (sph + sth - two * sqrt_fn(sph * sth)))

    cls = None
    for c in range(10, _D):
        dc = P[c] - T[c]
        sq = dc * dc
        cls = sq if cls is None else cls + sq

    l_obj = jnp.float32(_L_COORD) * loc + contain + notcontain + cls
    d0 = P[0] - T[0]
    d1 = P[1] - T[1]
    l_noobj = jnp.float32(_L_NOOBJ) * (d0 * d0 + d1 * d1)
    return jnp.where(T[0] > jnp.float32(0.0), l_obj, l_noobj)


def _sc_body(pred_hbm, tgt_hbm, out_hbm,
             pb0, pb1, tb0, tb1, accb,
             ps0, ps1, ts0, ts1):
    cid = lax.axis_index("c")
    sid = lax.axis_index("s")
    wid = sid * 2 + cid
    bt = wid // 2
    pos0 = (wid % 2) * _POS_PER_TILE
    b0 = pl.multiple_of(bt * _BT, _BT)

    pbufs = (pb0, pb1)
    tbufs = (tb0, tb1)
    psems = (ps0, ps1)
    tsems = (ts0, ts1)

    def start(k, b):
        p0 = pos0 + k * _CHUNK_POS
        for c in range(_D):
            pltpu.async_copy(
                pred_hbm.at[pl.ds(p0, _CHUNK_POS), pl.ds(c, 1), pl.ds(b0, _BT)],
                pbufs[b].at[pl.ds(c * _CHUNK_POS, _CHUNK_POS)], psems[b])
            pltpu.async_copy(
                tgt_hbm.at[pl.ds(p0, _CHUNK_POS), pl.ds(c, 1), pl.ds(b0, _BT)],
                tbufs[b].at[pl.ds(c * _CHUNK_POS, _CHUNK_POS)], tsems[b])

    def wait_chunk(b):
        for c in range(_D):
            pltpu.make_async_copy(
                pred_hbm.at[pl.ds(pos0, _CHUNK_POS), pl.ds(c, 1), pl.ds(b0, _BT)],
                pbufs[b].at[pl.ds(c * _CHUNK_POS, _CHUNK_POS)], psems[b]).wait()
            pltpu.make_async_copy(
                tgt_hbm.at[pl.ds(pos0, _CHUNK_POS), pl.ds(c, 1), pl.ds(b0, _BT)],
                tbufs[b].at[pl.ds(c * _CHUNK_POS, _CHUNK_POS)], tsems[b]).wait()

    def process(b, acc):
        pbuf = pbufs[b]
        tbuf = tbufs[b]

        def body(q, a):
            i = q // _LGROUPS
            l16 = (q - i * _LGROUPS) * 16
            P = [pbuf[c * _CHUNK_POS + i, 0, pl.ds(l16, 16)] for c in range(_D)]
            T = [tbuf[c * _CHUNK_POS + i, 0, pl.ds(l16, 16)] for c in range(_D)]
            return a + _cell_loss(P, T, _sqrt_newton)

        return lax.fori_loop(0, _CHUNK_POS * _LGROUPS, body, acc)

    start(0, 0)
    start(1, 1)

    def chunk_pair(kk, acc):
        for b in range(2):
            wait_chunk(b)
            acc = process(b, acc)
            k_next = 2 * kk + b + 2

            @pl.when(k_next < _NCHUNK)
            def _():
                start(k_next, b)
        return acc

    acc = lax.fori_loop(0, _NCHUNK // 2, chunk_pair,
                        jnp.zeros((16,), jnp.float32))

    accb[...] = acc
    pltpu.sync_copy(accb, out_hbm.at[wid])


_yolo_sc = functools.partial(
    pl.kernel,
    out_type=jax.ShapeDtypeStruct((_NW, 16), jnp.float32),
    mesh=plsc.VectorSubcoreMesh(core_axis_name="c", subcore_axis_name="s"),
    compiler_params=pltpu.CompilerParams(use_tc_tiling_on_sc=True,
                                         needs_layout_passes=False),
    scratch_types=[
        pltpu.VMEM((_CHUNK_POS * _D, 1, _BT), jnp.float32),
        pltpu.VMEM((_CHUNK_POS * _D, 1, _BT), jnp.float32),
        pltpu.VMEM((_CHUNK_POS * _D, 1, _BT), jnp.float32),
        pltpu.VMEM((_CHUNK_POS * _D, 1, _BT), jnp.float32),
        pltpu.VMEM((16,), jnp.float32),
        pltpu.SemaphoreType.DMA,
        pltpu.SemaphoreType.DMA,
        pltpu.SemaphoreType.DMA,
        pltpu.SemaphoreType.DMA,
    ],
)(_sc_body)


def _tc_body(pref, tref, oref):
    i = pl.program_id(0)
    p = pref[...]
    t = tref[...]
    P = [p[:, c, :] for c in range(_D)]
    T = [t[:, c, :] for c in range(_D)]
    partial = jnp.sum(_cell_loss(P, T, jnp.sqrt))

    @pl.when(i == 0)
    def _():
        oref[0, 0] = partial

    @pl.when(i > 0)
    def _():
        oref[0, 0] = oref[0, 0] + partial


_yolo_tc = pl.pallas_call(
    _tc_body,
    grid=(_P_TC // _PB,),
    in_specs=[
        pl.BlockSpec((_PB, _D, _BATCH), lambda i: (i + _P_SC // _PB, 0, 0)),
        pl.BlockSpec((_PB, _D, _BATCH), lambda i: (i + _P_SC // _PB, 0, 0)),
    ],
    out_specs=pl.BlockSpec(memory_space=pltpu.SMEM),
    out_shape=jax.ShapeDtypeStruct((1, 1), jnp.float32),
)


def kernel(pred_tensor, target_tensor):
    pt = jnp.transpose(pred_tensor, (1, 2, 3, 0)).reshape(_P, _D, _BATCH)
    tt = jnp.transpose(target_tensor, (1, 2, 3, 0)).reshape(_P, _D, _BATCH)
    sc_partials = _yolo_sc(pt, tt)
    tc_partial = _yolo_tc(pt, tt)
    total = jnp.sum(sc_partials) + tc_partial[0, 0]
    return total * jnp.float32(1.0 / _BATCH)


# speedup vs baseline: 1.4213x; 1.4213x over previous
import functools

import jax
import jax.numpy as jnp
import numpy as np
from jax import lax
from jax.experimental import pallas as pl
from jax.experimental.pallas import tpu as pltpu
from jax.experimental.pallas import tpu_sc as plsc

_BATCH = 2048
_S = 14
_D = 30
_P = _S * _S
_NW = 32
_BT = 128
_P_SC = 112
_P_TC = _P - _P_SC
_POS_PER_TILE = _P_SC // 2
_CHUNK_POS = 7
_NCHUNK = _POS_PER_TILE // _CHUNK_POS
_LGROUPS = _BT // 16
_PB = 7
_INV_S = 1.0 / _S
_L_COORD = 5.0
_L_NOOBJ = 0.5


def _sqrt_newton(x):
    i = lax.bitcast_convert_type(x, jnp.int32)
    y = lax.bitcast_convert_type(np.int32(0x5F3759DF) - (i >> 1), jnp.float32)
    xh = jnp.float32(0.5) * x
    three_half = jnp.float32(1.5)
    y = y * (three_half - xh * y * y)
    y = y * (three_half - xh * y * y)
    y = y * (three_half - xh * y * y)
    return x * y


def _cell_loss(P, T, sqrt_fn):
    inv_s = jnp.float32(_INV_S)
    half = jnp.float32(0.5)

    def corners(cx, cy, w, h):
        x = cx * inv_s
        y = cy * inv_s
        hw = half * w
        hh = half * h
        return x - hw, y - hh, x + hw, y + hh

    ax1, ay1, ax2, ay2 = corners(P[2], P[3], P[4], P[5])
    bx1, by1, bx2, by2 = corners(P[6], P[7], P[8], P[9])
    tx1, ty1, tx2, ty2 = corners(T[2], T[3], T[4], T[5])
    area_t = (tx2 - tx1) * (ty2 - ty1)

    def iou(x1, y1, x2, y2):
        lx = jnp.maximum(x1, tx1)
        ly = jnp.maximum(y1, ty1)
        rx = jnp.minimum(x2, tx2)
        ry = jnp.minimum(y2, ty2)
        iw = jnp.maximum(rx - lx, jnp.float32(0.0))
        ih = jnp.maximum(ry - ly, jnp.float32(0.0))
        inter = iw * ih
        area = (x2 - x1) * (y2 - y1)
        return inter / (area + area_t - inter + jnp.float32(1e-10))

    iou0 = iou(ax1, ay1, ax2, ay2)
    iou1 = iou(bx1, by1, bx2, by2)
    sel1 = iou1 > iou0
    max_iou = jnp.where(sel1, iou1, iou0)
    conf_a = jnp.where(sel1, P[1], P[0])
    conf_i = jnp.where(sel1, P[0], P[1])
    d_ca = conf_a - max_iou
    contain = d_ca * d_ca
    notcontain = conf_i * conf_i

    spx = jnp.where(sel1, P[6], P[2])
    spy = jnp.where(sel1, P[7], P[3])
    spw = jnp.where(sel1, P[8], P[4])
    sph = jnp.where(sel1, P[9], P[5])
    stx = jnp.where(sel1, T[6], T[2])
    sty = jnp.where(sel1, T[7], T[3])
    stw = jnp.where(sel1, T[8], T[4])
    sth = jnp.where(sel1, T[9], T[5])
    dx = spx - stx
    dy = spy - sty
    two = jnp.float32(2.0)
    loc = (dx * dx + dy * dy
           + (spw + stw - two * sqrt_fn(spw * stw))
           + (sph + sth - two * sqrt_fn(sph * sth)))

    cls = None
    for c in range(10, _D):
        dc = P[c] - T[c]
        sq = dc * dc
        cls = sq if cls is None else cls + sq

    l_obj = jnp.float32(_L_COORD) * loc + contain + notcontain + cls
    d0 = P[0] - T[0]
    d1 = P[1] - T[1]
    l_noobj = jnp.float32(_L_NOOBJ) * (d0 * d0 + d1 * d1)
    return jnp.where(T[0] > jnp.float32(0.0), l_obj, l_noobj)


def _sc_body(pred_hbm, tgt_hbm, out_hbm,
             pb0, pb1, tb0, tb1, accb,
             ps0, ps1, ts0, ts1):
    cid = lax.axis_index("c")
    sid = lax.axis_index("s")
    wid = sid * 2 + cid
    bt = wid // 2
    pos0 = (wid % 2) * _POS_PER_TILE
    b0 = pl.multiple_of(bt * _BT, _BT)

    pbufs = (pb0, pb1)
    tbufs = (tb0, tb1)
    psems = (ps0, ps1)
    tsems = (ts0, ts1)

    def start(k, b):
        p0 = pos0 + k * _CHUNK_POS
        for c in range(_D):
            pltpu.async_copy(
                pred_hbm.at[pl.ds(p0, _CHUNK_POS), pl.ds(c, 1), pl.ds(b0, _BT)],
                pbufs[b].at[pl.ds(c * _CHUNK_POS, _CHUNK_POS)], psems[b])
            pltpu.async_copy(
                tgt_hbm.at[pl.ds(p0, _CHUNK_POS), pl.ds(c, 1), pl.ds(b0, _BT)],
                tbufs[b].at[pl.ds(c * _CHUNK_POS, _CHUNK_POS)], tsems[b])

    def wait_chunk(b):
        for c in range(_D):
            pltpu.make_async_copy(
                pred_hbm.at[pl.ds(pos0, _CHUNK_POS), pl.ds(c, 1), pl.ds(b0, _BT)],
                pbufs[b].at[pl.ds(c * _CHUNK_POS, _CHUNK_POS)], psems[b]).wait()
            pltpu.make_async_copy(
                tgt_hbm.at[pl.ds(pos0, _CHUNK_POS), pl.ds(c, 1), pl.ds(b0, _BT)],
                tbufs[b].at[pl.ds(c * _CHUNK_POS, _CHUNK_POS)], tsems[b]).wait()

    def process(b, acc):
        pbuf = pbufs[b]
        tbuf = tbufs[b]

        def body(q, a):
            i = q // _LGROUPS
            l16 = (q - i * _LGROUPS) * 16
            P = [pbuf[c * _CHUNK_POS + i, 0, pl.ds(l16, 16)] for c in range(_D)]
            T = [tbuf[c * _CHUNK_POS + i, 0, pl.ds(l16, 16)] for c in range(_D)]
            return a + _cell_loss(P, T, _sqrt_newton)

        return lax.fori_loop(0, _CHUNK_POS * _LGROUPS, body, acc)

    start(0, 0)
    start(1, 1)

    def chunk_pair(kk, acc):
        for b in range(2):
            wait_chunk(b)
            acc = process(b, acc)
            k_next = 2 * kk + b + 2

            @pl.when(k_next < _NCHUNK)
            def _():
                start(k_next, b)
        return acc

    acc = lax.fori_loop(0, _NCHUNK // 2, chunk_pair,
                        jnp.zeros((16,), jnp.float32))

    accb[...] = acc
    pltpu.sync_copy(accb, out_hbm.at[wid])


_yolo_sc = functools.partial(
    pl.kernel,
    out_type=jax.ShapeDtypeStruct((_NW, 16), jnp.float32),
    mesh=plsc.VectorSubcoreMesh(core_axis_name="c", subcore_axis_name="s"),
    compiler_params=pltpu.CompilerParams(use_tc_tiling_on_sc=True,
                                         needs_layout_passes=False),
    scratch_types=[
        pltpu.VMEM((_CHUNK_POS * _D, 1, _BT), jnp.float32),
        pltpu.VMEM((_CHUNK_POS * _D, 1, _BT), jnp.float32),
        pltpu.VMEM((_CHUNK_POS * _D, 1, _BT), jnp.float32),
        pltpu.VMEM((_CHUNK_POS * _D, 1, _BT), jnp.float32),
        pltpu.VMEM((16,), jnp.float32),
        pltpu.SemaphoreType.DMA,
        pltpu.SemaphoreType.DMA,
        pltpu.SemaphoreType.DMA,
        pltpu.SemaphoreType.DMA,
    ],
)(_sc_body)


def _tc_body(pref, tref, oref):
    i = pl.program_id(0)
    p = pref[...]
    t = tref[...]
    d2 = (p - t) ** 2
    c_idx = lax.broadcasted_iota(jnp.int32, (1, _D, 1), 1)
    w_cls = jnp.where(c_idx >= 10, jnp.float32(1.0), jnp.float32(0.0))
    cls = jnp.sum(d2 * w_cls, axis=1)

    P = [p[:, c, :] for c in range(10)]
    T = [t[:, c, :] for c in range(10)]
    d0 = P[0] - T[0]
    d1 = P[1] - T[1]
    noobj2 = d0 * d0 + d1 * d1

    inv_s = jnp.float32(_INV_S)
    half = jnp.float32(0.5)

    def corners(cx, cy, w, h):
        x = cx * inv_s
        y = cy * inv_s
        hw = half * w
        hh = half * h
        return x - hw, y - hh, x + hw, y + hh

    ax1, ay1, ax2, ay2 = corners(P[2], P[3], P[4], P[5])
    bx1, by1, bx2, by2 = corners(P[6], P[7], P[8], P[9])
    tx1, ty1, tx2, ty2 = corners(T[2], T[3], T[4], T[5])
    area_t = (tx2 - tx1) * (ty2 - ty1)

    def iou(x1, y1, x2, y2):
        lx = jnp.maximum(x1, tx1)
        ly = jnp.maximum(y1, ty1)
        rx = jnp.minimum(x2, tx2)
        ry = jnp.minimum(y2, ty2)
        iw = jnp.maximum(rx - lx, jnp.float32(0.0))
        ih = jnp.maximum(ry - ly, jnp.float32(0.0))
        inter = iw * ih
        area = (x2 - x1) * (y2 - y1)
        return inter / (area + area_t - inter + jnp.float32(1e-10))

    iou0 = iou(ax1, ay1, ax2, ay2)
    iou1 = iou(bx1, by1, bx2, by2)
    sel1 = iou1 > iou0
    max_iou = jnp.where(sel1, iou1, iou0)
    conf_a = jnp.where(sel1, P[1], P[0])
    conf_i = jnp.where(sel1, P[0], P[1])
    d_ca = conf_a - max_iou
    contain = d_ca * d_ca
    notcontain = conf_i * conf_i

    spx = jnp.where(sel1, P[6], P[2])
    spy = jnp.where(sel1, P[7], P[3])
    spw = jnp.where(sel1, P[8], P[4])
    sph = jnp.where(sel1, P[9], P[5])
    stx = jnp.where(sel1, T[6], T[2])
    sty = jnp.where(sel1, T[7], T[3])
    stw = jnp.where(sel1, T[8], T[4])
    sth = jnp.where(sel1, T[9], T[5])
    dx = spx - stx
    dy = spy - sty
    two = jnp.float32(2.0)
    loc = (dx * dx + dy * dy
           + (spw + stw - two * jnp.sqrt(spw * stw))
           + (sph + sth - two * jnp.sqrt(sph * sth)))

    l_obj = jnp.float32(_L_COORD) * loc + contain + notcontain + cls
    l_noobj = jnp.float32(_L_NOOBJ) * noobj2
    cell = jnp.where(T[0] > jnp.float32(0.0), l_obj, l_noobj)
    partial = jnp.sum(cell)

    @pl.when(i == 0)
    def _():
        oref[0, 0] = partial

    @pl.when(i > 0)
    def _():
        oref[0, 0] = oref[0, 0] + partial


_yolo_tc = pl.pallas_call(
    _tc_body,
    grid=(_P_TC // _PB,),
    in_specs=[
        pl.BlockSpec((_PB, _D, _BATCH), lambda i: (i + _P_SC // _PB, 0, 0)),
        pl.BlockSpec((_PB, _D, _BATCH), lambda i: (i + _P_SC // _PB, 0, 0)),
    ],
    out_specs=pl.BlockSpec(memory_space=pltpu.SMEM),
    out_shape=jax.ShapeDtypeStruct((1, 1), jnp.float32),
)


def kernel(pred_tensor, target_tensor):
    pt = jnp.transpose(pred_tensor, (1, 2, 3, 0)).reshape(_P, _D, _BATCH)
    tt = jnp.transpose(target_tensor, (1, 2, 3, 0)).reshape(_P, _D, _BATCH)
    sc_partials = _yolo_sc(pt, tt)
    tc_partial = _yolo_tc(pt, tt)
    total = jnp.sum(sc_partials) + tc_partial[0, 0]
    return total * jnp.float32(1.0 / _BATCH)


# speedup vs baseline: 1.4856x; 1.0452x over previous
import functools

import jax
import jax.numpy as jnp
import numpy as np
from jax import lax
from jax.experimental import pallas as pl
from jax.experimental.pallas import tpu as pltpu
from jax.experimental.pallas import tpu_sc as plsc

_BATCH = 2048
_S = 14
_D = 30
_P = _S * _S
_NW = 32
_BT = 128
_P_SC = 126
_P_TC = _P - _P_SC
_POS_PER_TILE = _P_SC // 2
_CHUNK_POS = 7
_NCHUNK = _POS_PER_TILE // _CHUNK_POS
_LGROUPS = _BT // 16
_PB = 14
_INV_S = 1.0 / _S
_L_COORD = 5.0
_L_NOOBJ = 0.5


def _sqrt_newton(x):
    i = lax.bitcast_convert_type(x, jnp.int32)
    y = lax.bitcast_convert_type(np.int32(0x5F3759DF) - (i >> 1), jnp.float32)
    xh = jnp.float32(0.5) * x
    three_half = jnp.float32(1.5)
    y = y * (three_half - xh * y * y)
    y = y * (three_half - xh * y * y)
    y = y * (three_half - xh * y * y)
    return x * y


def _cell_loss(P, T, sqrt_fn):
    inv_s = jnp.float32(_INV_S)
    half = jnp.float32(0.5)

    def corners(cx, cy, w, h):
        x = cx * inv_s
        y = cy * inv_s
        hw = half * w
        hh = half * h
        return x - hw, y - hh, x + hw, y + hh

    ax1, ay1, ax2, ay2 = corners(P[2], P[3], P[4], P[5])
    bx1, by1, bx2, by2 = corners(P[6], P[7], P[8], P[9])
    tx1, ty1, tx2, ty2 = corners(T[2], T[3], T[4], T[5])
    area_t = (tx2 - tx1) * (ty2 - ty1)

    def iou(x1, y1, x2, y2):
        lx = jnp.maximum(x1, tx1)
        ly = jnp.maximum(y1, ty1)
        rx = jnp.minimum(x2, tx2)
        ry = jnp.minimum(y2, ty2)
        iw = jnp.maximum(rx - lx, jnp.float32(0.0))
        ih = jnp.maximum(ry - ly, jnp.float32(0.0))
        inter = iw * ih
        area = (x2 - x1) * (y2 - y1)
        return inter / (area + area_t - inter + jnp.float32(1e-10))

    iou0 = iou(ax1, ay1, ax2, ay2)
    iou1 = iou(bx1, by1, bx2, by2)
    sel1 = iou1 > iou0
    max_iou = jnp.where(sel1, iou1, iou0)
    conf_a = jnp.where(sel1, P[1], P[0])
    conf_i = jnp.where(sel1, P[0], P[1])
    d_ca = conf_a - max_iou
    contain = d_ca * d_ca
    notcontain = conf_i * conf_i

    spx = jnp.where(sel1, P[6], P[2])
    spy = jnp.where(sel1, P[7], P[3])
    spw = jnp.where(sel1, P[8], P[4])
    sph = jnp.where(sel1, P[9], P[5])
    stx = jnp.where(sel1, T[6], T[2])
    sty = jnp.where(sel1, T[7], T[3])
    stw = jnp.where(sel1, T[8], T[4])
    sth = jnp.where(sel1, T[9], T[5])
    dx = spx - stx
    dy = spy - sty
    two = jnp.float32(2.0)
    loc = (dx * dx + dy * dy
           + (spw + stw - two * sqrt_fn(spw * stw))
           + (sph + sth - two * sqrt_fn(sph * sth)))

    cls = None
    for c in range(10, _D):
        dc = P[c] - T[c]
        sq = dc * dc
        cls = sq if cls is None else cls + sq

    l_obj = jnp.float32(_L_COORD) * loc + contain + notcontain + cls
    d0 = P[0] - T[0]
    d1 = P[1] - T[1]
    l_noobj = jnp.float32(_L_NOOBJ) * (d0 * d0 + d1 * d1)
    return jnp.where(T[0] > jnp.float32(0.0), l_obj, l_noobj)


def _sc_body(pred_hbm, tgt_hbm, out_hbm,
             pb0, pb1, tb0, tb1, accb,
             ps0, ps1, ts0, ts1):
    cid = lax.axis_index("c")
    sid = lax.axis_index("s")
    wid = sid * 2 + cid
    bt = wid // 2
    pos0 = (wid % 2) * _POS_PER_TILE
    b0 = pl.multiple_of(bt * _BT, _BT)

    pbufs = (pb0, pb1)
    tbufs = (tb0, tb1)
    psems = (ps0, ps1)
    tsems = (ts0, ts1)

    def start(k, b):
        p0 = pos0 + k * _CHUNK_POS
        for c in range(_D):
            pltpu.async_copy(
                pred_hbm.at[pl.ds(p0, _CHUNK_POS), pl.ds(c, 1), pl.ds(b0, _BT)],
                pbufs[b].at[pl.ds(c * _CHUNK_POS, _CHUNK_POS)], psems[b])
            pltpu.async_copy(
                tgt_hbm.at[pl.ds(p0, _CHUNK_POS), pl.ds(c, 1), pl.ds(b0, _BT)],
                tbufs[b].at[pl.ds(c * _CHUNK_POS, _CHUNK_POS)], tsems[b])

    def wait_chunk(b):
        for c in range(_D):
            pltpu.make_async_copy(
                pred_hbm.at[pl.ds(pos0, _CHUNK_POS), pl.ds(c, 1), pl.ds(b0, _BT)],
                pbufs[b].at[pl.ds(c * _CHUNK_POS, _CHUNK_POS)], psems[b]).wait()
            pltpu.make_async_copy(
                tgt_hbm.at[pl.ds(pos0, _CHUNK_POS), pl.ds(c, 1), pl.ds(b0, _BT)],
                tbufs[b].at[pl.ds(c * _CHUNK_POS, _CHUNK_POS)], tsems[b]).wait()

    def process(b, acc):
        pbuf = pbufs[b]
        tbuf = tbufs[b]

        def body(q, a):
            i = q // _LGROUPS
            l16 = (q - i * _LGROUPS) * 16
            P = [pbuf[c * _CHUNK_POS + i, 0, pl.ds(l16, 16)] for c in range(_D)]
            T = [tbuf[c * _CHUNK_POS + i, 0, pl.ds(l16, 16)] for c in range(_D)]
            return a + _cell_loss(P, T, _sqrt_newton)

        return lax.fori_loop(0, _CHUNK_POS * _LGROUPS, body, acc)

    start(0, 0)
    start(1, 1)

    def chunk_pair(kk, acc):
        for b in range(2):
            wait_chunk(b)
            acc = process(b, acc)
            k_next = 2 * kk + b + 2

            @pl.when(k_next < _NCHUNK)
            def _():
                start(k_next, b)
        return acc

    acc = lax.fori_loop(0, _NCHUNK // 2, chunk_pair,
                        jnp.zeros((16,), jnp.float32))
    if _NCHUNK % 2:
        wait_chunk(0)
        acc = process(0, acc)

    accb[...] = acc
    pltpu.sync_copy(accb, out_hbm.at[wid])


_yolo_sc = functools.partial(
    pl.kernel,
    out_type=jax.ShapeDtypeStruct((_NW, 16), jnp.float32),
    mesh=plsc.VectorSubcoreMesh(core_axis_name="c", subcore_axis_name="s"),
    compiler_params=pltpu.CompilerParams(use_tc_tiling_on_sc=True,
                                         needs_layout_passes=False),
    scratch_types=[
        pltpu.VMEM((_CHUNK_POS * _D, 1, _BT), jnp.float32),
        pltpu.VMEM((_CHUNK_POS * _D, 1, _BT), jnp.float32),
        pltpu.VMEM((_CHUNK_POS * _D, 1, _BT), jnp.float32),
        pltpu.VMEM((_CHUNK_POS * _D, 1, _BT), jnp.float32),
        pltpu.VMEM((16,), jnp.float32),
        pltpu.SemaphoreType.DMA,
        pltpu.SemaphoreType.DMA,
        pltpu.SemaphoreType.DMA,
        pltpu.SemaphoreType.DMA,
    ],
)(_sc_body)


def _tc_body(pref, tref, oref):
    i = pl.program_id(0)
    p = pref[...]
    t = tref[...]
    d2 = (p - t) ** 2
    c_idx = lax.broadcasted_iota(jnp.int32, (1, _D, 1), 1)
    w_cls = jnp.where(c_idx >= 10, jnp.float32(1.0), jnp.float32(0.0))
    cls = jnp.sum(d2 * w_cls, axis=1)

    P = [p[:, c, :] for c in range(10)]
    T = [t[:, c, :] for c in range(10)]
    d0 = P[0] - T[0]
    d1 = P[1] - T[1]
    noobj2 = d0 * d0 + d1 * d1

    inv_s = jnp.float32(_INV_S)
    half = jnp.float32(0.5)

    def corners(cx, cy, w, h):
        x = cx * inv_s
        y = cy * inv_s
        hw = half * w
        hh = half * h
        return x - hw, y - hh, x + hw, y + hh

    ax1, ay1, ax2, ay2 = corners(P[2], P[3], P[4], P[5])
    bx1, by1, bx2, by2 = corners(P[6], P[7], P[8], P[9])
    tx1, ty1, tx2, ty2 = corners(T[2], T[3], T[4], T[5])
    area_t = (tx2 - tx1) * (ty2 - ty1)

    def iou(x1, y1, x2, y2):
        lx = jnp.maximum(x1, tx1)
        ly = jnp.maximum(y1, ty1)
        rx = jnp.minimum(x2, tx2)
        ry = jnp.minimum(y2, ty2)
        iw = jnp.maximum(rx - lx, jnp.float32(0.0))
        ih = jnp.maximum(ry - ly, jnp.float32(0.0))
        inter = iw * ih
        area = (x2 - x1) * (y2 - y1)
        return inter / (area + area_t - inter + jnp.float32(1e-10))

    iou0 = iou(ax1, ay1, ax2, ay2)
    iou1 = iou(bx1, by1, bx2, by2)
    sel1 = iou1 > iou0
    max_iou = jnp.where(sel1, iou1, iou0)
    conf_a = jnp.where(sel1, P[1], P[0])
    conf_i = jnp.where(sel1, P[0], P[1])
    d_ca = conf_a - max_iou
    contain = d_ca * d_ca
    notcontain = conf_i * conf_i

    spx = jnp.where(sel1, P[6], P[2])
    spy = jnp.where(sel1, P[7], P[3])
    spw = jnp.where(sel1, P[8], P[4])
    sph = jnp.where(sel1, P[9], P[5])
    stx = jnp.where(sel1, T[6], T[2])
    sty = jnp.where(sel1, T[7], T[3])
    stw = jnp.where(sel1, T[8], T[4])
    sth = jnp.where(sel1, T[9], T[5])
    dx = spx - stx
    dy = spy - sty
    two = jnp.float32(2.0)
    loc = (dx * dx + dy * dy
           + (spw + stw - two * jnp.sqrt(spw * stw))
           + (sph + sth - two * jnp.sqrt(sph * sth)))

    l_obj = jnp.float32(_L_COORD) * loc + contain + notcontain + cls
    l_noobj = jnp.float32(_L_NOOBJ) * noobj2
    cell = jnp.where(T[0] > jnp.float32(0.0), l_obj, l_noobj)
    partial = jnp.sum(cell)

    @pl.when(i == 0)
    def _():
        oref[0, 0] = partial

    @pl.when(i > 0)
    def _():
        oref[0, 0] = oref[0, 0] + partial


_yolo_tc = pl.pallas_call(
    _tc_body,
    grid=(_P_TC // _PB,),
    in_specs=[
        pl.BlockSpec((_PB, _D, _BATCH), lambda i: (i + _P_SC // _PB, 0, 0)),
        pl.BlockSpec((_PB, _D, _BATCH), lambda i: (i + _P_SC // _PB, 0, 0)),
    ],
    out_specs=pl.BlockSpec(memory_space=pltpu.SMEM),
    out_shape=jax.ShapeDtypeStruct((1, 1), jnp.float32),
)


def kernel(pred_tensor, target_tensor):
    pt = jnp.transpose(pred_tensor, (1, 2, 3, 0)).reshape(_P, _D, _BATCH)
    tt = jnp.transpose(target_tensor, (1, 2, 3, 0)).reshape(_P, _D, _BATCH)
    sc_partials = _yolo_sc(pt, tt)
    tc_partial = _yolo_tc(pt, tt)
    total = jnp.sum(sc_partials) + tc_partial[0, 0]
    return total * jnp.float32(1.0 / _BATCH)


# speedup vs baseline: 1.4994x; 1.0093x over previous
import functools

import jax
import jax.numpy as jnp
import numpy as np
from jax import lax
from jax.experimental import pallas as pl
from jax.experimental.pallas import tpu as pltpu
from jax.experimental.pallas import tpu_sc as plsc

_BATCH = 2048
_S = 14
_D = 30
_P = _S * _S
_NW = 32
_BT = 128
_P_SC = 126
_P_TC = _P - _P_SC
_POS_PER_TILE = _P_SC // 2
_CHUNK_POS = 7
_NCHUNK = _POS_PER_TILE // _CHUNK_POS
_LGROUPS = _BT // 16
_PB = 7
_INV_S = 1.0 / _S
_L_COORD = 5.0
_L_NOOBJ = 0.5


def _sqrt_newton(x):
    i = lax.bitcast_convert_type(x, jnp.int32)
    y = lax.bitcast_convert_type(np.int32(0x5F3759DF) - (i >> 1), jnp.float32)
    xh = jnp.float32(0.5) * x
    three_half = jnp.float32(1.5)
    y = y * (three_half - xh * y * y)
    y = y * (three_half - xh * y * y)
    y = y * (three_half - xh * y * y)
    return x * y


def _cell_loss(P, T, sqrt_fn):
    inv_s = jnp.float32(_INV_S)
    half = jnp.float32(0.5)

    def corners(cx, cy, w, h):
        x = cx * inv_s
        y = cy * inv_s
        hw = half * w
        hh = half * h
        return x - hw, y - hh, x + hw, y + hh

    ax1, ay1, ax2, ay2 = corners(P[2], P[3], P[4], P[5])
    bx1, by1, bx2, by2 = corners(P[6], P[7], P[8], P[9])
    tx1, ty1, tx2, ty2 = corners(T[2], T[3], T[4], T[5])
    area_t = (tx2 - tx1) * (ty2 - ty1)

    def iou(x1, y1, x2, y2):
        lx = jnp.maximum(x1, tx1)
        ly = jnp.maximum(y1, ty1)
        rx = jnp.minimum(x2, tx2)
        ry = jnp.minimum(y2, ty2)
        iw = jnp.maximum(rx - lx, jnp.float32(0.0))
        ih = jnp.maximum(ry - ly, jnp.float32(0.0))
        inter = iw * ih
        area = (x2 - x1) * (y2 - y1)
        return inter / (area + area_t - inter + jnp.float32(1e-10))

    iou0 = iou(ax1, ay1, ax2, ay2)
    iou1 = iou(bx1, by1, bx2, by2)
    sel1 = iou1 > iou0
    max_iou = jnp.where(sel1, iou1, iou0)
    conf_a = jnp.where(sel1, P[1], P[0])
    conf_i = jnp.where(sel1, P[0], P[1])
    d_ca = conf_a - max_iou
    contain = d_ca * d_ca
    notcontain = conf_i * conf_i

    spx = jnp.where(sel1, P[6], P[2])
    spy = jnp.where(sel1, P[7], P[3])
    spw = jnp.where(sel1, P[8], P[4])
    sph = jnp.where(sel1, P[9], P[5])
    stx = jnp.where(sel1, T[6], T[2])
    sty = jnp.where(sel1, T[7], T[3])
    stw = jnp.where(sel1, T[8], T[4])
    sth = jnp.where(sel1, T[9], T[5])
    dx = spx - stx
    dy = spy - sty
    two = jnp.float32(2.0)
    loc = (dx * dx + dy * dy
           + (spw + stw - two * sqrt_fn(spw * stw))
           + (sph + sth - two * sqrt_fn(sph * sth)))

    cls = None
    for c in range(10, _D):
        dc = P[c] - T[c]
        sq = dc * dc
        cls = sq if cls is None else cls + sq

    l_obj = jnp.float32(_L_COORD) * loc + contain + notcontain + cls
    d0 = P[0] - T[0]
    d1 = P[1] - T[1]
    l_noobj = jnp.float32(_L_NOOBJ) * (d0 * d0 + d1 * d1)
    return jnp.where(T[0] > jnp.float32(0.0), l_obj, l_noobj)


def _sc_body(pred_hbm, tgt_hbm, out_hbm,
             pb0, pb1, tb0, tb1, accb,
             ps0, ps1, ts0, ts1):
    cid = lax.axis_index("c")
    sid = lax.axis_index("s")
    wid = sid * 2 + cid
    bt = wid // 2
    pos0 = (wid % 2) * _POS_PER_TILE
    b0 = pl.multiple_of(bt * _BT, _BT)

    pbufs = (pb0, pb1)
    tbufs = (tb0, tb1)
    psems = (ps0, ps1)
    tsems = (ts0, ts1)

    def start(k, b):
        p0 = pos0 + k * _CHUNK_POS
        for c in range(_D):
            pltpu.async_copy(
                pred_hbm.at[pl.ds(p0, _CHUNK_POS), pl.ds(c, 1), pl.ds(b0, _BT)],
                pbufs[b].at[pl.ds(c * _CHUNK_POS, _CHUNK_POS)], psems[b])
            pltpu.async_copy(
                tgt_hbm.at[pl.ds(p0, _CHUNK_POS), pl.ds(c, 1), pl.ds(b0, _BT)],
                tbufs[b].at[pl.ds(c * _CHUNK_POS, _CHUNK_POS)], tsems[b])

    def wait_chunk(b):
        for c in range(_D):
            pltpu.make_async_copy(
                pred_hbm.at[pl.ds(pos0, _CHUNK_POS), pl.ds(c, 1), pl.ds(b0, _BT)],
                pbufs[b].at[pl.ds(c * _CHUNK_POS, _CHUNK_POS)], psems[b]).wait()
            pltpu.make_async_copy(
                tgt_hbm.at[pl.ds(pos0, _CHUNK_POS), pl.ds(c, 1), pl.ds(b0, _BT)],
                tbufs[b].at[pl.ds(c * _CHUNK_POS, _CHUNK_POS)], tsems[b]).wait()

    def process(b, acc):
        pbuf = pbufs[b]
        tbuf = tbufs[b]

        def body(q, a):
            i = q // _LGROUPS
            l16 = (q - i * _LGROUPS) * 16
            P = [pbuf[c * _CHUNK_POS + i, 0, pl.ds(l16, 16)] for c in range(_D)]
            T = [tbuf[c * _CHUNK_POS + i, 0, pl.ds(l16, 16)] for c in range(_D)]
            return a + _cell_loss(P, T, _sqrt_newton)

        return lax.fori_loop(0, _CHUNK_POS * _LGROUPS, body, acc)

    start(0, 0)
    start(1, 1)

    def chunk_pair(kk, acc):
        for b in range(2):
            wait_chunk(b)
            acc = process(b, acc)
            k_next = 2 * kk + b + 2

            @pl.when(k_next < _NCHUNK)
            def _():
                start(k_next, b)
        return acc

    acc = lax.fori_loop(0, _NCHUNK // 2, chunk_pair,
                        jnp.zeros((16,), jnp.float32))
    if _NCHUNK % 2:
        wait_chunk(0)
        acc = process(0, acc)

    accb[...] = acc
    pltpu.sync_copy(accb, out_hbm.at[wid])


_yolo_sc = functools.partial(
    pl.kernel,
    out_type=jax.ShapeDtypeStruct((_NW, 16), jnp.float32),
    mesh=plsc.VectorSubcoreMesh(core_axis_name="c", subcore_axis_name="s"),
    compiler_params=pltpu.CompilerParams(use_tc_tiling_on_sc=True,
                                         needs_layout_passes=False),
    scratch_types=[
        pltpu.VMEM((_CHUNK_POS * _D, 1, _BT), jnp.float32),
        pltpu.VMEM((_CHUNK_POS * _D, 1, _BT), jnp.float32),
        pltpu.VMEM((_CHUNK_POS * _D, 1, _BT), jnp.float32),
        pltpu.VMEM((_CHUNK_POS * _D, 1, _BT), jnp.float32),
        pltpu.VMEM((16,), jnp.float32),
        pltpu.SemaphoreType.DMA,
        pltpu.SemaphoreType.DMA,
        pltpu.SemaphoreType.DMA,
        pltpu.SemaphoreType.DMA,
    ],
)(_sc_body)


def _tc_body(pref, tref, oref):
    i = pl.program_id(0)
    p = pref[...]
    t = tref[...]
    d2 = (p - t) ** 2
    c_idx = lax.broadcasted_iota(jnp.int32, (1, _D, 1), 1)
    w_cls = jnp.where(c_idx >= 10, jnp.float32(1.0), jnp.float32(0.0))
    cls = jnp.sum(d2 * w_cls, axis=1)

    P = [p[:, c, :] for c in range(10)]
    T = [t[:, c, :] for c in range(10)]
    d0 = P[0] - T[0]
    d1 = P[1] - T[1]
    noobj2 = d0 * d0 + d1 * d1

    inv_s = jnp.float32(_INV_S)
    half = jnp.float32(0.5)

    def corners(cx, cy, w, h):
        x = cx * inv_s
        y = cy * inv_s
        hw = half * w
        hh = half * h
        return x - hw, y - hh, x + hw, y + hh

    ax1, ay1, ax2, ay2 = corners(P[2], P[3], P[4], P[5])
    bx1, by1, bx2, by2 = corners(P[6], P[7], P[8], P[9])
    tx1, ty1, tx2, ty2 = corners(T[2], T[3], T[4], T[5])
    area_t = (tx2 - tx1) * (ty2 - ty1)

    def iou(x1, y1, x2, y2):
        lx = jnp.maximum(x1, tx1)
        ly = jnp.maximum(y1, ty1)
        rx = jnp.minimum(x2, tx2)
        ry = jnp.minimum(y2, ty2)
        iw = jnp.maximum(rx - lx, jnp.float32(0.0))
        ih = jnp.maximum(ry - ly, jnp.float32(0.0))
        inter = iw * ih
        area = (x2 - x1) * (y2 - y1)
        return inter / (area + area_t - inter + jnp.float32(1e-10))

    iou0 = iou(ax1, ay1, ax2, ay2)
    iou1 = iou(bx1, by1, bx2, by2)
    sel1 = iou1 > iou0
    max_iou = jnp.where(sel1, iou1, iou0)
    conf_a = jnp.where(sel1, P[1], P[0])
    conf_i = jnp.where(sel1, P[0], P[1])
    d_ca = conf_a - max_iou
    contain = d_ca * d_ca
    notcontain = conf_i * conf_i

    spx = jnp.where(sel1, P[6], P[2])
    spy = jnp.where(sel1, P[7], P[3])
    spw = jnp.where(sel1, P[8], P[4])
    sph = jnp.where(sel1, P[9], P[5])
    stx = jnp.where(sel1, T[6], T[2])
    sty = jnp.where(sel1, T[7], T[3])
    stw = jnp.where(sel1, T[8], T[4])
    sth = jnp.where(sel1, T[9], T[5])
    dx = spx - stx
    dy = spy - sty
    two = jnp.float32(2.0)
    loc = (dx * dx + dy * dy
           + (spw + stw - two * jnp.sqrt(spw * stw))
           + (sph + sth - two * jnp.sqrt(sph * sth)))

    l_obj = jnp.float32(_L_COORD) * loc + contain + notcontain + cls
    l_noobj = jnp.float32(_L_NOOBJ) * noobj2
    cell = jnp.where(T[0] > jnp.float32(0.0), l_obj, l_noobj)
    partial = jnp.sum(cell)

    @pl.when(i == 0)
    def _():
        oref[0, 0] = partial

    @pl.when(i > 0)
    def _():
        oref[0, 0] = oref[0, 0] + partial


_yolo_tc = pl.pallas_call(
    _tc_body,
    grid=(_P_TC // _PB,),
    in_specs=[
        pl.BlockSpec((_PB, _D, _BATCH), lambda i: (i + _P_SC // _PB, 0, 0)),
        pl.BlockSpec((_PB, _D, _BATCH), lambda i: (i + _P_SC // _PB, 0, 0)),
    ],
    out_specs=pl.BlockSpec(memory_space=pltpu.SMEM),
    out_shape=jax.ShapeDtypeStruct((1, 1), jnp.float32),
)


def kernel(pred_tensor, target_tensor):
    pt = jnp.transpose(pred_tensor, (1, 2, 3, 0)).reshape(_P, _D, _BATCH)
    tt = jnp.transpose(target_tensor, (1, 2, 3, 0)).reshape(_P, _D, _BATCH)
    sc_partials = _yolo_sc(pt, tt)
    tc_partial = _yolo_tc(pt, tt)
    total = jnp.sum(sc_partials) + tc_partial[0, 0]
    return total * jnp.float32(1.0 / _BATCH)
